# Initial kernel scaffold; baseline (speedup 1.0000x reference)
import jax
import jax.numpy as jnp
from jax.experimental import pallas as pl


def kernel(z, embedding):
    raise NotImplementedError("write your pallas kernel here")


# speedup vs baseline: 27.1675x; 27.1675x over previous
import jax
import jax.numpy as jnp
from jax.experimental import pallas as pl
from jax.experimental.pallas import tpu as pltpu

_NE = 1024
_ED = 256
_B = 16
_H = 576
_BETA = 0.25


def _vq_body(z_ref, e_ref, zsq_ref, esq_ref,
             zq_ref, idx_ref, loss_ref, ppl_ref,
             counts_ref, acc_ref):
    b = pl.program_id(0)
    emb = e_ref[...]
    zb = z_ref[0]

    mm = jax.lax.dot_general(emb, zb, (((1,), (0,)), ((), ())),
                             preferred_element_type=jnp.float32)
    d = (-zsq_ref[0] - esq_ref[...]) + 2.0 * mm

    m = jnp.max(d, axis=0, keepdims=True)
    iota = jax.lax.broadcasted_iota(jnp.int32, (_NE, _H), 0)
    idx = jnp.max(jnp.where(d == m, iota, -1), axis=0)
    idx_ref[0, 0] = idx

    onehot = (iota == idx[None, :]).astype(jnp.float32)
    zq = jax.lax.dot_general(emb, onehot, (((0,), (0,)), ((), ())),
                             preferred_element_type=jnp.float32)
    zq_ref[0] = zq

    cnt = jnp.sum(onehot, axis=1, keepdims=True)
    msum = jnp.sum(m, axis=1, keepdims=True)

    @pl.when(b == 0)
    def _init():
        counts_ref[...] = cnt
        acc_ref[...] = msum

    @pl.when(b > 0)
    def _accum():
        counts_ref[...] += cnt
        acc_ref[...] += msum

    @pl.when(b == _B - 1)
    def _finalize():
        loss_ref[...] = (-(1.0 + _BETA) / (_B * _H * _ED)) * acc_ref[...]
        p = counts_ref[...] * (1.0 / (_B * _H))
        ppl_ref[...] = jnp.exp(-jnp.sum(p * jnp.log(p + 1e-10),
                                        axis=0, keepdims=True))


def kernel(z, embedding):
    z_flat = jnp.transpose(z, (0, 2, 1)).reshape(-1, _ED)
    zsq = jnp.sum(z_flat ** 2, axis=1).reshape(_B, 1, _H)
    esq = jnp.sum(embedding ** 2, axis=1).reshape(_NE, 1)

    zq, idx3, loss, ppl = pl.pallas_call(
        _vq_body,
        grid=(_B,),
        in_specs=[
            pl.BlockSpec((1, _ED, _H), lambda b: (b, 0, 0)),
            pl.BlockSpec((_NE, _ED), lambda b: (0, 0)),
            pl.BlockSpec((1, 1, _H), lambda b: (b, 0, 0)),
            pl.BlockSpec((_NE, 1), lambda b: (0, 0)),
        ],
        out_specs=[
            pl.BlockSpec((1, _ED, _H), lambda b: (b, 0, 0)),
            pl.BlockSpec((1, 1, _H), lambda b: (b, 0, 0)),
            pl.BlockSpec((1, 1), lambda b: (0, 0)),
            pl.BlockSpec((1, 1), lambda b: (0, 0)),
        ],
        out_shape=[
            jax.ShapeDtypeStruct((_B, _ED, _H), jnp.float32),
            jax.ShapeDtypeStruct((_B, 1, _H), jnp.int32),
            jax.ShapeDtypeStruct((1, 1), jnp.float32),
            jax.ShapeDtypeStruct((1, 1), jnp.float32),
        ],
        scratch_shapes=[
            pltpu.VMEM((_NE, 1), jnp.float32),
            pltpu.VMEM((1, 1), jnp.float32),
        ],
        compiler_params=pltpu.CompilerParams(
            dimension_semantics=("arbitrary",)),
    )(z, embedding, zsq, esq)

    return (zq, loss[0, 0], idx3.reshape(_B, _H), ppl[0, 0])
